# Initial kernel scaffold; baseline (speedup 1.0000x reference)
import jax
import jax.numpy as jnp
from jax.experimental import pallas as pl


def kernel(scalar_features, cartesian_pos, edge_index, msg_W, msg_b, msg_g, msg_beta, upd_W, upd_b, upd_g, upd_beta):
    raise NotImplementedError("write your pallas kernel here")


# speedup vs baseline: 2.5987x; 2.5987x over previous
import functools

import jax
import jax.numpy as jnp
from jax import lax
from jax.experimental import pallas as pl
from jax.experimental.pallas import tpu as pltpu
from jax.experimental.pallas import tpu_sc as plsc

_N = 10000
_E = 320000
_D = 128
_BG = 128
_EP = 327680
_NP = 10112
_BE = 512
_BN = 2000
_NS = 16
_NPT = _NP // _NS

_HI = lax.Precision.HIGHEST
_DOT = (((1,), (0,)), ((), ()))


def _erf(x):
    ax = jnp.abs(x)
    t = 1.0 / (1.0 + 0.3275911 * ax)
    poly = ((((1.061405429 * t - 1.453152027) * t + 1.421413741) * t
             - 0.284496736) * t + 0.254829592) * t
    y = 1.0 - poly * jnp.exp(-ax * ax)
    return jnp.sign(x) * y


def _gelu(x):
    return 0.5 * x * (1.0 + _erf(x * 0.7071067811865476))


def _ln(x, g, b):
    m = jnp.mean(x, axis=-1, keepdims=True)
    c = x - m
    v = jnp.mean(c * c, axis=-1, keepdims=True)
    return c * lax.rsqrt(v + 1e-5) * g + b


def _pq_call(h, wr, wc, b2d):
    def body(h_ref, wr_ref, wc_ref, b_ref, p_ref, q_ref):
        hh = h_ref[...]
        p_ref[...] = lax.dot_general(hh, wr_ref[...], _DOT,
                                     preferred_element_type=jnp.float32,
                                     precision=_HI) + b_ref[...]
        q_ref[...] = lax.dot_general(hh, wc_ref[...], _DOT,
                                     preferred_element_type=jnp.float32,
                                     precision=_HI)
    return pl.pallas_call(
        body,
        grid=(_N // _BN,),
        in_specs=[
            pl.BlockSpec((_BN, _D), lambda i: (i, 0)),
            pl.BlockSpec((_D, _D), lambda i: (0, 0)),
            pl.BlockSpec((_D, _D), lambda i: (0, 0)),
            pl.BlockSpec((1, _D), lambda i: (0, 0)),
        ],
        out_specs=[pl.BlockSpec((_BN, _D), lambda i: (i, 0)),
                   pl.BlockSpec((_BN, _D), lambda i: (i, 0))],
        out_shape=[jax.ShapeDtypeStruct((_N, _D), jnp.float32)] * 2,
    )(h, wr, wc, b2d)


def _msg0_call(g1, g2, c1, c2, wd, g, beta):
    def body(g1_ref, g2_ref, c1_ref, c2_ref, wd_ref, g_ref, b_ref, m_ref, d_ref):
        diff = c1_ref[...] - c2_ref[...]
        d2 = jnp.sum(diff * diff, axis=1, keepdims=True)
        dist = jnp.sqrt(d2)
        pre = g1_ref[...] + g2_ref[...] + dist * wd_ref[...]
        m_ref[...] = _ln(_gelu(pre), g_ref[...], b_ref[...])
        d_ref[...] = jnp.broadcast_to(dist, (_BE, 16))
    return pl.pallas_call(
        body,
        grid=(_EP // _BE,),
        in_specs=[
            pl.BlockSpec((_BE, _D), lambda i: (i, 0)),
            pl.BlockSpec((_BE, _D), lambda i: (i, 0)),
            pl.BlockSpec((_BE, _D), lambda i: (i, 0)),
            pl.BlockSpec((_BE, _D), lambda i: (i, 0)),
            pl.BlockSpec((1, _D), lambda i: (0, 0)),
            pl.BlockSpec((1, _D), lambda i: (0, 0)),
            pl.BlockSpec((1, _D), lambda i: (0, 0)),
        ],
        out_specs=[pl.BlockSpec((_BE, _D), lambda i: (i, 0)),
                   pl.BlockSpec((_BE, 16), lambda i: (i, 0))],
        out_shape=[jax.ShapeDtypeStruct((_EP, _D), jnp.float32),
                   jax.ShapeDtypeStruct((_EP, 16), jnp.float32)],
    )(g1, g2, c1, c2, wd, g, beta)


def _msg_call(g1, g2, dist, wd, g, beta):
    def body(g1_ref, g2_ref, d_ref, wd_ref, g_ref, b_ref, m_ref):
        pre = g1_ref[...] + g2_ref[...] + d_ref[:, :1] * wd_ref[...]
        m_ref[...] = _ln(_gelu(pre), g_ref[...], b_ref[...])
    return pl.pallas_call(
        body,
        grid=(_EP // _BE,),
        in_specs=[
            pl.BlockSpec((_BE, _D), lambda i: (i, 0)),
            pl.BlockSpec((_BE, _D), lambda i: (i, 0)),
            pl.BlockSpec((_BE, 16), lambda i: (i, 0)),
            pl.BlockSpec((1, _D), lambda i: (0, 0)),
            pl.BlockSpec((1, _D), lambda i: (0, 0)),
            pl.BlockSpec((1, _D), lambda i: (0, 0)),
        ],
        out_specs=pl.BlockSpec((_BE, _D), lambda i: (i, 0)),
        out_shape=jax.ShapeDtypeStruct((_EP, _D), jnp.float32),
    )(g1, g2, dist, wd, g, beta)


def _upd_call(h, p0, p1, c0, c1, wu1, wu2, b2d, g2d, beta2d):
    def body(h_ref, p0_ref, p1_ref, c0_ref, c1_ref, w1_ref, w2_ref,
             b_ref, g_ref, be_ref, o_ref):
        cnt = c0_ref[:, :1] + c1_ref[:, :1]
        agg = (p0_ref[...] + p1_ref[...]) / jnp.maximum(cnt, 1.0)
        hh = h_ref[...]
        u = (lax.dot_general(hh, w1_ref[...], _DOT,
                             preferred_element_type=jnp.float32, precision=_HI)
             + lax.dot_general(agg, w2_ref[...], _DOT,
                               preferred_element_type=jnp.float32, precision=_HI)
             + b_ref[...])
        o_ref[...] = hh + _ln(u, g_ref[...], be_ref[...])
    return pl.pallas_call(
        body,
        grid=(_N // _BN,),
        in_specs=[
            pl.BlockSpec((_BN, _D), lambda i: (i, 0)),
            pl.BlockSpec((_BN, _D), lambda i: (i, 0)),
            pl.BlockSpec((_BN, _D), lambda i: (i, 0)),
            pl.BlockSpec((_BN, 16), lambda i: (i, 0)),
            pl.BlockSpec((_BN, 16), lambda i: (i, 0)),
            pl.BlockSpec((_D, _D), lambda i: (0, 0)),
            pl.BlockSpec((_D, _D), lambda i: (0, 0)),
            pl.BlockSpec((1, _D), lambda i: (0, 0)),
            pl.BlockSpec((1, _D), lambda i: (0, 0)),
            pl.BlockSpec((1, _D), lambda i: (0, 0)),
        ],
        out_specs=pl.BlockSpec((_BN, _D), lambda i: (i, 0)),
        out_shape=jax.ShapeDtypeStruct((_N, _D), jnp.float32),
    )(h, p0, p1, c0, c1, wu1, wu2, b2d, g2d, beta2d)


_MESH = dict(core_axis_name="c", subcore_axis_name="s")


def _sc_gather1(tab, idx):
    d = tab.shape[1]
    e = idx.shape[1]
    mesh = plsc.VectorSubcoreMesh(**_MESH)

    @functools.partial(
        pl.kernel, mesh=mesh,
        out_type=jax.ShapeDtypeStruct((e, d), jnp.float32),
    )
    def k(t_hbm, i_hbm, o_hbm):
        def body(i_v, o_v):
            pltpu.sync_copy(t_hbm.at[i_v.at[0]], o_v)
        pltpu.emit_pipeline(
            body, grid=(e // _BG,),
            in_specs=[pl.BlockSpec((1, _BG), lambda i: (0, i))],
            out_specs=[pl.BlockSpec((_BG, d), lambda i: (i, 0))],
            core_axis_name=("c", "s"),
            dimension_semantics=(pltpu.PARALLEL,),
        )(i_hbm, o_hbm)

    return k(tab, idx)


def _sc_gather2(tab1, tab2, idx1, idx2):
    return _sc_gather1(tab1, idx1), _sc_gather1(tab2, idx2)


def _sc_scatter_add(m, col2d):
    mesh = plsc.VectorSubcoreMesh(**_MESH)

    @functools.partial(
        pl.kernel, mesh=mesh,
        out_type=jax.ShapeDtypeStruct((2 * _NP, _D), jnp.float32),
        scratch_types=[pltpu.VMEM_SHARED((_NP, _D), jnp.float32)])
    def k(m_hbm, c_hbm, z_hbm, part_hbm, acc_sh):
        core = lax.axis_index("c")
        sid = lax.axis_index("s")
        r0 = sid * _NPT
        pltpu.sync_copy(z_hbm.at[pl.ds(r0, _NPT)], acc_sh.at[pl.ds(r0, _NPT)])
        plsc.subcore_barrier()

        def body(m_v, c_v):
            pltpu.sync_copy(m_v, acc_sh.at[c_v.at[0]], add=True)

        pltpu.emit_pipeline(
            body, grid=(_EP // _BG,),
            in_specs=[pl.BlockSpec((_BG, _D), lambda i: (i, 0)),
                      pl.BlockSpec((1, _BG), lambda i: (0, i))],
            out_specs=[],
            core_axis_name=("c", "s"),
            dimension_semantics=(pltpu.PARALLEL,),
        )(m_hbm, c_hbm)
        plsc.subcore_barrier()
        pltpu.sync_copy(acc_sh.at[pl.ds(r0, _NPT)],
                        part_hbm.at[pl.ds(core * _NP + r0, _NPT)])

    return k(m, col2d, jnp.zeros((_NP, _D), jnp.float32))


def _sc_count(col2d):
    mesh = plsc.VectorSubcoreMesh(**_MESH)

    @functools.partial(
        pl.kernel, mesh=mesh,
        out_type=jax.ShapeDtypeStruct((2 * _NP, _D), jnp.float32),
        scratch_types=[pltpu.VMEM_SHARED((_NP, _D), jnp.float32),
                       pltpu.VMEM((_BG, _D), jnp.float32)])
    def k(c_hbm, zc_hbm, ones_hbm, cpart_hbm, acc_c_sh, ones_v):
        core = lax.axis_index("c")
        sid = lax.axis_index("s")
        r0 = sid * _NPT
        pltpu.sync_copy(zc_hbm.at[pl.ds(r0, _NPT)], acc_c_sh.at[pl.ds(r0, _NPT)])
        pltpu.sync_copy(ones_hbm, ones_v)
        plsc.subcore_barrier()

        def body(c_v):
            pltpu.sync_copy(ones_v, acc_c_sh.at[c_v.at[0]], add=True)

        pltpu.emit_pipeline(
            body, grid=(_EP // _BG,),
            in_specs=[pl.BlockSpec((1, _BG), lambda i: (0, i))],
            out_specs=[],
            core_axis_name=("c", "s"),
            dimension_semantics=(pltpu.PARALLEL,),
        )(c_hbm)
        plsc.subcore_barrier()
        pltpu.sync_copy(acc_c_sh.at[pl.ds(r0, _NPT)],
                        cpart_hbm.at[pl.ds(core * _NP + r0, _NPT)])

    return k(col2d, jnp.zeros((_NP, _D), jnp.float32),
             jnp.ones((_BG, _D), jnp.float32))


def kernel(scalar_features, cartesian_pos, edge_index, msg_W, msg_b, msg_g,
           msg_beta, upd_W, upd_b, upd_g, upd_beta):
    h = scalar_features
    pad = _EP - _E
    row2d = jnp.concatenate(
        [edge_index[0:1], jnp.zeros((1, pad), jnp.int32)], axis=1)
    col2d = jnp.concatenate(
        [edge_index[1:2], jnp.full((1, pad), _N, jnp.int32)], axis=1)
    cartpad = jnp.pad(cartesian_pos, ((0, 0), (0, _D - 3)))
    c1, c2 = _sc_gather2(cartpad, cartpad, row2d, col2d)

    dist = None
    cnts = None
    for i in range(3):
        wr = msg_W[i, :_D]
        wc = msg_W[i, _D:2 * _D]
        wd = msg_W[i, 2 * _D:]
        p, q = _pq_call(h, wr, wc, msg_b[i:i + 1])
        g1, g2 = _sc_gather2(p, q, row2d, col2d)
        if i == 0:
            m, dist = _msg0_call(g1, g2, c1, c2, wd, msg_g[i:i + 1],
                                 msg_beta[i:i + 1])
            cnts = _sc_count(col2d)[:, :16]
        else:
            m = _msg_call(g1, g2, dist, wd, msg_g[i:i + 1], msg_beta[i:i + 1])
        parts = _sc_scatter_add(m, col2d)
        h = _upd_call(h, parts[:_N], parts[_NP:_NP + _N],
                      cnts[:_N], cnts[_NP:_NP + _N],
                      upd_W[i, :_D], upd_W[i, _D:], upd_b[i:i + 1],
                      upd_g[i:i + 1], upd_beta[i:i + 1])
    return h
